# baseline (speedup 1.0000x reference)
import functools

import jax
import jax.numpy as jnp
from jax import lax
from jax.experimental import pallas as pl
from jax.experimental.pallas import tpu as pltpu
from jax.experimental.pallas import tpu_sc as plsc

B = 16384
D = 64
NC = 2
NS = 16
NW = NC * NS
B_PER_W = B // NW
CHUNK = 128
NCHUNK = B_PER_W // CHUNK

ENT_N = 1000000
CT = 256
NFULL = ENT_N // CT
TAIL = ENT_N - NFULL * CT
OUT_ROWS = ENT_N // 2


def _transpose_kernel(entT, tail2, out, in0, in1, ou0, ou1, tout_v,
                      si0, si1, so0, so1):
    wid = lax.axis_index("s") * NC + lax.axis_index("c")
    iota = lax.iota(jnp.int32, 16)
    nit = jnp.where(wid < NFULL % NW, NFULL // NW + 1, NFULL // NW)
    ins = (in0, in1)
    outs = (ou0, ou1)
    isem = (si0, si1)
    osem = (so0, so1)

    full = iota < 16
    tjs = [(iota + j) & 15 for j in range(16)]
    rshs = [t >> 1 for t in tjs]
    couts = [[(tjs[j] & 1) * 64 + db * 16 + iota for j in range(16)]
             for db in range(4)]
    rins = [db * 16 + iota for db in range(4)]

    def transpose(in_b, out_b):
        def do_cb(cb, _):
            for j in range(16):
                col_in = cb * 16 + tjs[j]
                row_out = cb * 8 + rshs[j]
                for db in range(4):
                    v = plsc.load_gather(in_b, [rins[db], col_in],
                                         mask=full)
                    plsc.store_scatter(out_b, [row_out, couts[db][j]], v,
                                       mask=full)
            return 0

        lax.fori_loop(0, CT // 16, do_cb, 0, unroll=2)

    pltpu.async_copy(entT.at[:, pl.ds(wid * CT, CT)], in0, si0)

    def pair(ip, _):
        for ph in range(2):
            it = 2 * ip + ph

            @pl.when(it < nit)
            def _body():
                ci = wid + it * NW

                @pl.when(it + 1 < nit)
                def _prefetch():
                    cn = wid + (it + 1) * NW
                    pltpu.async_copy(entT.at[:, pl.ds(cn * CT, CT)],
                                     ins[1 - ph], isem[1 - ph])

                pltpu.make_async_copy(
                    entT.at[:, pl.ds(ci * CT, CT)], ins[ph],
                    isem[ph]).wait()

                @pl.when(it >= 2)
                def _drain_prev():
                    pltpu.make_async_copy(
                        outs[ph], out.at[pl.ds(0, CT // 2), :],
                        osem[ph]).wait()

                transpose(ins[ph], outs[ph])
                pltpu.async_copy(outs[ph],
                                 out.at[pl.ds(ci * (CT // 2), CT // 2), :],
                                 osem[ph])
        return 0

    lax.fori_loop(0, (nit + 1) // 2, pair, 0)
    pltpu.make_async_copy(ou0, out.at[pl.ds(0, CT // 2), :], so0).wait()
    pltpu.make_async_copy(ou1, out.at[pl.ds(0, CT // 2), :], so1).wait()

    @pl.when(wid == NW - 1)
    def _tail():
        pltpu.sync_copy(tail2, tout_v)
        pltpu.sync_copy(tout_v,
                        out.at[pl.ds(NFULL * (CT // 2), TAIL // 2), :])


def _score_kernel(r0, r1, r2, r3, r4, r5, p0, p1, p2, p3, p4, p5,
                  ent2, rel2, tim2, out_hbm,
                  idx_v, par_v, b0, b1, b2, b3, b4, b5, pos_v, neg_v, sem):
    wid = lax.axis_index("s") * NC + lax.axis_index("c")
    base = wid * B_PER_W
    lanes = lax.iota(jnp.int32, 16)
    perms = [lanes ^ s for s in (1, 2, 4, 8)]
    lane0 = lanes == 0

    def lane_sum(v):
        for p in perms:
            v = v + v.at[p].get(mode="promise_in_bounds")
        return v

    row_srcs = (r0, r1, r2, r3, r4, r5)
    par_srcs = (p0, p1, p2, p3, p4, p5)
    tables = (ent2, ent2, rel2, ent2, ent2, tim2)
    bufs = (b0, b1, b2, b3, b4, b5)

    for c in range(NCHUNK):
        off = base + c * CHUNK
        for j in range(6):
            pltpu.sync_copy(row_srcs[j].at[pl.ds(off, CHUNK)], idx_v.at[j])
            pltpu.sync_copy(par_srcs[j].at[pl.ds(off, CHUNK)],
                            par_v.at[j, pl.ds(0, CHUNK)])
        cps = [pltpu.async_copy(tables[j].at[idx_v.at[j]], bufs[j], sem)
               for j in range(6)]
        for cp in cps:
            cp.wait()

        def body(e, _):
            offs = [par_v[j, pl.ds(e, 16)][0] for j in range(6)]
            ip_p = jnp.zeros((16,), jnp.float32)
            ip_n = jnp.zeros((16,), jnp.float32)
            dps, dns, ts = [], [], []
            for k in range(4):
                h = b0[e, pl.ds(offs[0] + 16 * k, 16)]
                tl = b1[e, pl.ds(offs[1] + 16 * k, 16)]
                r = b2[e, pl.ds(offs[2] + 16 * k, 16)]
                nh = b3[e, pl.ds(offs[3] + 16 * k, 16)]
                nt = b4[e, pl.ds(offs[4] + 16 * k, 16)]
                t = b5[e, pl.ds(offs[5] + 16 * k, 16)]
                dp = h + r - tl
                dn = nh + r - nt
                ip_p = ip_p + dp * t
                ip_n = ip_n + dn * t
                dps.append(dp)
                dns.append(dn)
                ts.append(t)
            sp = lane_sum(ip_p)
            sn = lane_sum(ip_n)
            ap = jnp.zeros((16,), jnp.float32)
            an = jnp.zeros((16,), jnp.float32)
            for k in range(4):
                ap = ap + jnp.abs(dps[k] - ts[k] * sp)
                an = an + jnp.abs(dns[k] - ts[k] * sn)
            eidx = jnp.full((16,), e, jnp.int32)
            plsc.store_scatter(pos_v, [eidx], lane_sum(ap), mask=lane0)
            plsc.store_scatter(neg_v, [eidx], lane_sum(an), mask=lane0)
            return 0

        lax.fori_loop(0, CHUNK, body, 0, unroll=4)
        pltpu.sync_copy(pos_v, out_hbm.at[0, pl.ds(off, CHUNK)])
        pltpu.sync_copy(neg_v, out_hbm.at[1, pl.ds(off, CHUNK)])


@jax.jit
def _run(ph, pt, rl, nh, nt, yr, ent, rel, time):
    mesh = plsc.VectorSubcoreMesh(core_axis_name="c", subcore_axis_name="s")
    cp = pltpu.CompilerParams(needs_layout_passes=False)

    tfn = functools.partial(
        pl.kernel,
        mesh=mesh,
        compiler_params=cp,
        out_type=jax.ShapeDtypeStruct((OUT_ROWS, 128), jnp.float32),
        scratch_types=[
            pltpu.VMEM((64, CT), jnp.float32),
            pltpu.VMEM((64, CT), jnp.float32),
            pltpu.VMEM((CT // 2, 128), jnp.float32),
            pltpu.VMEM((CT // 2, 128), jnp.float32),
            pltpu.VMEM((TAIL // 2, 128), jnp.float32),
            pltpu.SemaphoreType.DMA,
            pltpu.SemaphoreType.DMA,
            pltpu.SemaphoreType.DMA,
            pltpu.SemaphoreType.DMA,
        ],
    )(_transpose_kernel)
    tail2 = ent[NFULL * CT:, :].reshape(TAIL // 2, 128)
    ent2 = tfn(ent.T, tail2)

    sfn = functools.partial(
        pl.kernel,
        mesh=mesh,
        compiler_params=cp,
        out_type=jax.ShapeDtypeStruct((2, B), jnp.float32),
        scratch_types=[
            pltpu.VMEM((6, CHUNK), jnp.int32),
            pltpu.VMEM((6, CHUNK + 16), jnp.int32),
            pltpu.VMEM((CHUNK, 128), jnp.float32),
            pltpu.VMEM((CHUNK, 128), jnp.float32),
            pltpu.VMEM((CHUNK, 128), jnp.float32),
            pltpu.VMEM((CHUNK, 128), jnp.float32),
            pltpu.VMEM((CHUNK, 128), jnp.float32),
            pltpu.VMEM((CHUNK, 128), jnp.float32),
            pltpu.VMEM((CHUNK,), jnp.float32),
            pltpu.VMEM((CHUNK,), jnp.float32),
            pltpu.SemaphoreType.DMA,
        ],
    )(_score_kernel)

    rel2 = rel.reshape(-1, 2 * D)
    tim2 = time.reshape(-1, 2 * D)
    idxs = (ph, pt, rl, nh, nt, yr)
    rows = [i >> 1 for i in idxs]
    pars = [(i & 1) << 6 for i in idxs]
    return sfn(*rows, *pars, ent2, rel2, tim2)


def kernel(pos_head, pos_tail, rel, neg_head, neg_tail, start_year,
           ent_embeddings, rel_embeddings, time_embeddings):
    ph = pos_head.reshape(B)
    pt = pos_tail.reshape(B)
    rl = rel.reshape(B)
    nh = neg_head.reshape(B)
    nt = neg_tail.reshape(B)
    return _run(ph, pt, rl, nh, nt, start_year,
                ent_embeddings, rel_embeddings, time_embeddings)


# speedup vs baseline: 1.0054x; 1.0054x over previous
import functools

import jax
import jax.numpy as jnp
from jax import lax
from jax.experimental import pallas as pl
from jax.experimental.pallas import tpu as pltpu
from jax.experimental.pallas import tpu_sc as plsc

B = 16384
D = 64
NC = 2
NS = 16
NW = NC * NS
B_PER_W = B // NW
CHUNK = 128
NCHUNK = B_PER_W // CHUNK

ENT_N = 1000000
CT = 256
NFULL = ENT_N // CT
TAIL = ENT_N - NFULL * CT
OUT_ROWS = ENT_N // 2


def _transpose_kernel(entT, tail2, out, in0, in1, ou0, ou1, tout_v,
                      si0, si1, so0, so1):
    wid = lax.axis_index("s") * NC + lax.axis_index("c")
    iota = lax.iota(jnp.int32, 16)
    nit = jnp.where(wid < NFULL % NW, NFULL // NW + 1, NFULL // NW)
    ins = (in0, in1)
    outs = (ou0, ou1)
    isem = (si0, si1)
    osem = (so0, so1)

    full = iota < 16
    tjs = [(iota + j) & 15 for j in range(16)]
    rshs = [t >> 1 for t in tjs]
    couts = [[(tjs[j] & 1) * 64 + db * 16 + iota for j in range(16)]
             for db in range(4)]
    rins = [db * 16 + iota for db in range(4)]

    def transpose(in_b, out_b):
        def do_cb(cb, _):
            for j in range(16):
                col_in = cb * 16 + tjs[j]
                row_out = cb * 8 + rshs[j]
                for db in range(4):
                    v = plsc.load_gather(in_b, [rins[db], col_in],
                                         mask=full)
                    plsc.store_scatter(out_b, [row_out, couts[db][j]], v,
                                       mask=full)
            return 0

        lax.fori_loop(0, CT // 16, do_cb, 0, unroll=2)

    pltpu.async_copy(entT.at[:, pl.ds(wid * CT, CT)], in0, si0)

    def pair(ip, _):
        for ph in range(2):
            it = 2 * ip + ph

            @pl.when(it < nit)
            def _body():
                ci = wid + it * NW

                @pl.when(it + 1 < nit)
                def _prefetch():
                    cn = wid + (it + 1) * NW
                    pltpu.async_copy(entT.at[:, pl.ds(cn * CT, CT)],
                                     ins[1 - ph], isem[1 - ph])

                pltpu.make_async_copy(
                    entT.at[:, pl.ds(ci * CT, CT)], ins[ph],
                    isem[ph]).wait()

                @pl.when(it >= 2)
                def _drain_prev():
                    pltpu.make_async_copy(
                        outs[ph], out.at[pl.ds(0, CT // 2), :],
                        osem[ph]).wait()

                transpose(ins[ph], outs[ph])
                pltpu.async_copy(outs[ph],
                                 out.at[pl.ds(ci * (CT // 2), CT // 2), :],
                                 osem[ph])
        return 0

    lax.fori_loop(0, (nit + 1) // 2, pair, 0)
    pltpu.make_async_copy(ou0, out.at[pl.ds(0, CT // 2), :], so0).wait()
    pltpu.make_async_copy(ou1, out.at[pl.ds(0, CT // 2), :], so1).wait()

    @pl.when(wid == NW - 1)
    def _tail():
        pltpu.sync_copy(tail2, tout_v)
        pltpu.sync_copy(tout_v,
                        out.at[pl.ds(NFULL * (CT // 2), TAIL // 2), :])


def _score_kernel(r0, r1, r2, r3, r4, r5, p0, p1, p2, p3, p4, p5,
                  ent2, rel2, tim2, out_hbm,
                  idx_v, par_v, b0, b1, b2, b3, b4, b5, pos_v, neg_v, sem):
    wid = lax.axis_index("s") * NC + lax.axis_index("c")
    base = wid * B_PER_W
    lanes = lax.iota(jnp.int32, 16)
    perms = [lanes ^ s for s in (1, 2, 4, 8)]
    lane0 = lanes == 0

    def lane_sum(v):
        for p in perms:
            v = v + v.at[p].get(mode="promise_in_bounds")
        return v

    row_srcs = (r0, r1, r2, r3, r4, r5)
    par_srcs = (p0, p1, p2, p3, p4, p5)
    tables = (ent2, ent2, rel2, ent2, ent2, tim2)
    bufs = (b0, b1, b2, b3, b4, b5)

    for c in range(NCHUNK):
        off = base + c * CHUNK
        for j in range(6):
            pltpu.sync_copy(row_srcs[j].at[pl.ds(off, CHUNK)], idx_v.at[j])
            pltpu.sync_copy(par_srcs[j].at[pl.ds(off, CHUNK)],
                            par_v.at[j, pl.ds(0, CHUNK)])
        cps = [pltpu.async_copy(tables[j].at[idx_v.at[j]], bufs[j], sem)
               for j in range(6)]
        for cp in cps:
            cp.wait()

        def body(e, _):
            offs = [par_v[j, pl.ds(e, 16)][0] for j in range(6)]
            ip_p = jnp.zeros((16,), jnp.float32)
            ip_n = jnp.zeros((16,), jnp.float32)
            dps, dns, ts = [], [], []
            for k in range(4):
                h = b0[e, pl.ds(offs[0] + 16 * k, 16)]
                tl = b1[e, pl.ds(offs[1] + 16 * k, 16)]
                r = b2[e, pl.ds(offs[2] + 16 * k, 16)]
                nh = b3[e, pl.ds(offs[3] + 16 * k, 16)]
                nt = b4[e, pl.ds(offs[4] + 16 * k, 16)]
                t = b5[e, pl.ds(offs[5] + 16 * k, 16)]
                dp = h + r - tl
                dn = nh + r - nt
                ip_p = ip_p + dp * t
                ip_n = ip_n + dn * t
                dps.append(dp)
                dns.append(dn)
                ts.append(t)
            sp = lane_sum(ip_p)
            sn = lane_sum(ip_n)
            ap = jnp.zeros((16,), jnp.float32)
            an = jnp.zeros((16,), jnp.float32)
            for k in range(4):
                ap = ap + jnp.abs(dps[k] - ts[k] * sp)
                an = an + jnp.abs(dns[k] - ts[k] * sn)
            eidx = jnp.full((16,), e, jnp.int32)
            plsc.store_scatter(pos_v, [eidx], lane_sum(ap), mask=lane0)
            plsc.store_scatter(neg_v, [eidx], lane_sum(an), mask=lane0)
            return 0

        lax.fori_loop(0, CHUNK, body, 0, unroll=2)
        pltpu.sync_copy(pos_v, out_hbm.at[0, pl.ds(off, CHUNK)])
        pltpu.sync_copy(neg_v, out_hbm.at[1, pl.ds(off, CHUNK)])


@jax.jit
def _run(ph, pt, rl, nh, nt, yr, ent, rel, time):
    mesh = plsc.VectorSubcoreMesh(core_axis_name="c", subcore_axis_name="s")
    cp = pltpu.CompilerParams(needs_layout_passes=False)

    tfn = functools.partial(
        pl.kernel,
        mesh=mesh,
        compiler_params=cp,
        out_type=jax.ShapeDtypeStruct((OUT_ROWS, 128), jnp.float32),
        scratch_types=[
            pltpu.VMEM((64, CT), jnp.float32),
            pltpu.VMEM((64, CT), jnp.float32),
            pltpu.VMEM((CT // 2, 128), jnp.float32),
            pltpu.VMEM((CT // 2, 128), jnp.float32),
            pltpu.VMEM((TAIL // 2, 128), jnp.float32),
            pltpu.SemaphoreType.DMA,
            pltpu.SemaphoreType.DMA,
            pltpu.SemaphoreType.DMA,
            pltpu.SemaphoreType.DMA,
        ],
    )(_transpose_kernel)
    tail2 = ent[NFULL * CT:, :].reshape(TAIL // 2, 128)
    ent2 = tfn(ent.T, tail2)

    sfn = functools.partial(
        pl.kernel,
        mesh=mesh,
        compiler_params=cp,
        out_type=jax.ShapeDtypeStruct((2, B), jnp.float32),
        scratch_types=[
            pltpu.VMEM((6, CHUNK), jnp.int32),
            pltpu.VMEM((6, CHUNK + 16), jnp.int32),
            pltpu.VMEM((CHUNK, 128), jnp.float32),
            pltpu.VMEM((CHUNK, 128), jnp.float32),
            pltpu.VMEM((CHUNK, 128), jnp.float32),
            pltpu.VMEM((CHUNK, 128), jnp.float32),
            pltpu.VMEM((CHUNK, 128), jnp.float32),
            pltpu.VMEM((CHUNK, 128), jnp.float32),
            pltpu.VMEM((CHUNK,), jnp.float32),
            pltpu.VMEM((CHUNK,), jnp.float32),
            pltpu.SemaphoreType.DMA,
        ],
    )(_score_kernel)

    rel2 = rel.reshape(-1, 2 * D)
    tim2 = time.reshape(-1, 2 * D)
    idxs = (ph, pt, rl, nh, nt, yr)
    rows = [i >> 1 for i in idxs]
    pars = [(i & 1) << 6 for i in idxs]
    return sfn(*rows, *pars, ent2, rel2, tim2)


def kernel(pos_head, pos_tail, rel, neg_head, neg_tail, start_year,
           ent_embeddings, rel_embeddings, time_embeddings):
    ph = pos_head.reshape(B)
    pt = pos_tail.reshape(B)
    rl = rel.reshape(B)
    nh = neg_head.reshape(B)
    nt = neg_tail.reshape(B)
    return _run(ph, pt, rl, nh, nt, start_year,
                ent_embeddings, rel_embeddings, time_embeddings)
